# baseline (speedup 1.0000x reference)
import functools

import jax
import jax.numpy as jnp
from jax import lax
from jax.experimental import pallas as pl
from jax.experimental.pallas import tpu as pltpu
from jax.experimental.pallas import tpu_sc as plsc

_NC = 2
_NS = 16
_LANES = 16
_HALF = _NS // 2
_NGROUPS = _NC * _HALF
_NW = _NC * _NS


@functools.lru_cache(maxsize=None)
def _build_tc(B, GS, blk=2048):
    def body(gs_ref, w_ref, b_ref, o_ref):
        o_ref[...] = (
            jnp.dot(gs_ref[...], w_ref[...],
                    preferred_element_type=jnp.float32)
            + b_ref[...])

    return pl.pallas_call(
        body,
        grid=(B // blk,),
        in_specs=[
            pl.BlockSpec((blk, GS), lambda i: (i, 0)),
            pl.BlockSpec((GS, 1), lambda i: (0, 0)),
            pl.BlockSpec((1, 1), lambda i: (0, 0)),
        ],
        out_specs=pl.BlockSpec((blk, 1), lambda i: (i, 0)),
        out_shape=jax.ShapeDtypeStruct((B, 1), jnp.float32),
    )


@functools.lru_cache(maxsize=None)
def _build_sc(B, L, NP):
    bpg = B // _NGROUPS
    nch = bpg // _LANES
    bpt = B // _NW
    ncho = bpt // _LANES

    mesh = plsc.VectorSubcoreMesh(
        core_axis_name="c", subcore_axis_name="s",
        num_cores=_NC, num_subcores=_NS)

    @functools.partial(
        pl.kernel,
        out_type=jax.ShapeDtypeStruct((B,), jnp.float32),
        mesh=mesh,
        compiler_params=pltpu.CompilerParams(needs_layout_passes=False),
        scratch_types=[
            pltpu.VMEM((NP,), jnp.float32),
            pltpu.VMEM((L * bpg,), jnp.int32),
            pltpu.VMEM((bpg,), jnp.float32),
            pltpu.VMEM((bpt,), jnp.float32),
            pltpu.VMEM((bpt,), jnp.float32),
            pltpu.VMEM((bpt,), jnp.float32),
            pltpu.VMEM_SHARED((_NS * bpg,), jnp.float32),
            pltpu.SemaphoreType.DMA,
            pltpu.SemaphoreType.DMA,
        ],
    )
    def run(tabs_h, ids_h, p_h, out_h,
            tab_v, ids_v, psum_v, po_v, pd_v, pp_v, shared, sem, sem2):
        c = lax.axis_index("c")
        s = lax.axis_index("s")
        is_off = s < _HALF
        g = lax.rem(s, _HALF)
        base = (c * _HALF + g) * bpg
        obase = c * _HALF * bpg + s * bpt

        toffs = jnp.where(is_off, 0, NP)
        ioffs = jnp.where(is_off, 0, L * B)

        pltpu.async_copy(tabs_h.at[pl.ds(toffs, NP)], tab_v, sem)
        pltpu.async_copy(ids_h.at[pl.ds(ioffs + base * L, L * bpg)],
                         ids_v, sem2)
        pltpu.sync_copy(p_h.at[pl.ds(obase, bpt)], pp_v)

        iota5 = lax.iota(jnp.int32, _LANES) * L

        pltpu.make_async_copy(
            ids_h.at[pl.ds(ioffs + base * L, L * bpg)], ids_v, sem2).wait()
        pltpu.make_async_copy(tabs_h.at[pl.ds(toffs, NP)], tab_v, sem).wait()

        def chunk(i, carry):
            ib = i * (_LANES * L)
            acc = jnp.zeros((_LANES,), jnp.float32)
            for l in range(L):
                ids16 = plsc.load_gather(ids_v, [iota5 + (ib + l)])
                acc = acc + plsc.load_gather(tab_v, [ids16])
            psum_v[pl.ds(pl.multiple_of(i * _LANES, _LANES), _LANES)] = acc
            return carry

        lax.fori_loop(0, nch, chunk, 0)

        pltpu.sync_copy(psum_v, shared.at[pl.ds(s * bpg, bpg)])
        plsc.subcore_barrier()

        go = s // 2
        ho = lax.rem(s, 2) * bpt
        pltpu.sync_copy(shared.at[pl.ds(go * bpg + ho, bpt)], po_v)
        pltpu.sync_copy(shared.at[pl.ds((_HALF + go) * bpg + ho, bpt)], pd_v)

        def fold(i, carry):
            col = pl.multiple_of(i * _LANES, _LANES)
            po_v[pl.ds(col, _LANES)] = (
                po_v[pl.ds(col, _LANES)] + pd_v[pl.ds(col, _LANES)]
                + pp_v[pl.ds(col, _LANES)])
            return carry

        lax.fori_loop(0, ncho, fold, 0)
        pltpu.sync_copy(po_v, out_h.at[pl.ds(obase, bpt)])

    return run


def kernel(offense_ids, defense_ids, offense_pos, defense_pos, gamestate,
           off_embed, def_embed, bias, gs_w):
    del offense_pos, defense_pos
    B, L = offense_ids.shape
    NP = off_embed.shape[0]
    GS = gamestate.shape[1]
    p = _build_tc(B, GS)(
        gamestate.astype(jnp.float32),
        gs_w.astype(jnp.float32),
        bias.reshape(1, 1).astype(jnp.float32)).reshape(-1)
    tabs = jnp.concatenate(
        [off_embed.reshape(-1), def_embed.reshape(-1)]).astype(jnp.float32)
    ids = jnp.concatenate(
        [offense_ids.reshape(-1), defense_ids.reshape(-1)]).astype(jnp.int32)
    return _build_sc(B, L, NP)(tabs, ids, p)


# speedup vs baseline: 1.7489x; 1.7489x over previous
import functools

import jax
import jax.numpy as jnp
from jax import lax
from jax.experimental import pallas as pl
from jax.experimental.pallas import tpu as pltpu
from jax.experimental.pallas import tpu_sc as plsc

_NC = 2
_NS = 16
_LANES = 16
_HALF = _NS // 2
_NGROUPS = _NC * _HALF
_NW = _NC * _NS


@functools.lru_cache(maxsize=None)
def _build_matvec(B, GS, blk=4096):
    def body(gs_ref, w_ref, b_ref, o_ref):
        o_ref[...] = (
            jnp.dot(gs_ref[...], w_ref[...],
                    preferred_element_type=jnp.float32)
            + b_ref[...])[:, 0]

    return pl.pallas_call(
        body,
        grid=(B // blk,),
        in_specs=[
            pl.BlockSpec((blk, GS), lambda i: (i, 0)),
            pl.BlockSpec((GS, 1), lambda i: (0, 0)),
            pl.BlockSpec((1, 1), lambda i: (0, 0)),
        ],
        out_specs=pl.BlockSpec((blk,), lambda i: (i,)),
        out_shape=jax.ShapeDtypeStruct((B,), jnp.float32),
    )


@functools.lru_cache(maxsize=None)
def _build_combine(B):
    def body(g_ref, p_ref, o_ref):
        o_ref[...] = g_ref[...] + p_ref[...]

    return pl.pallas_call(
        body,
        out_shape=jax.ShapeDtypeStruct((B,), jnp.float32),
    )


@functools.lru_cache(maxsize=None)
def _build_sc(B, L, NP):
    bpg = B // _NGROUPS
    nch = bpg // _LANES
    bpt = B // _NW
    ncho = bpt // _LANES

    mesh = plsc.VectorSubcoreMesh(
        core_axis_name="c", subcore_axis_name="s",
        num_cores=_NC, num_subcores=_NS)

    @functools.partial(
        pl.kernel,
        out_type=jax.ShapeDtypeStruct((B,), jnp.float32),
        mesh=mesh,
        compiler_params=pltpu.CompilerParams(needs_layout_passes=False),
        scratch_types=[
            pltpu.VMEM((NP,), jnp.float32),
            pltpu.VMEM((L * bpg,), jnp.int32),
            pltpu.VMEM((bpg,), jnp.float32),
            pltpu.VMEM((bpt,), jnp.float32),
            pltpu.VMEM((bpt,), jnp.float32),
            pltpu.VMEM_SHARED((_NS * bpg,), jnp.float32),
            pltpu.SemaphoreType.DMA,
            pltpu.SemaphoreType.DMA,
        ],
    )
    def run(tabs_h, ids_h, out_h,
            tab_v, ids_v, psum_v, po_v, pd_v, shared, sem, sem2):
        c = lax.axis_index("c")
        s = lax.axis_index("s")
        is_off = s < _HALF
        g = lax.rem(s, _HALF)
        base = (c * _HALF + g) * bpg
        obase = c * _HALF * bpg + s * bpt

        toffs = jnp.where(is_off, 0, NP)
        ioffs = jnp.where(is_off, 0, L * B)

        pltpu.async_copy(tabs_h.at[pl.ds(toffs, NP)], tab_v, sem)
        for l in range(L):
            pltpu.async_copy(ids_h.at[pl.ds(ioffs + l * B + base, bpg)],
                             ids_v.at[pl.ds(l * bpg, bpg)], sem2)
        for l in range(L):
            pltpu.make_async_copy(
                ids_h.at[pl.ds(ioffs + l * B + base, bpg)],
                ids_v.at[pl.ds(l * bpg, bpg)], sem2).wait()
        pltpu.make_async_copy(tabs_h.at[pl.ds(toffs, NP)], tab_v, sem).wait()

        def chunk(i, carry):
            col = pl.multiple_of(i * _LANES, _LANES)
            acc = plsc.load_gather(tab_v, [ids_v[pl.ds(col, _LANES)]])
            for l in range(1, L):
                acc = acc + plsc.load_gather(
                    tab_v, [ids_v[pl.ds(l * bpg + col, _LANES)]])
            psum_v[pl.ds(col, _LANES)] = acc
            return carry

        lax.fori_loop(0, nch, chunk, 0)

        pltpu.sync_copy(psum_v, shared.at[pl.ds(s * bpg, bpg)])
        plsc.subcore_barrier()

        go = s // 2
        ho = lax.rem(s, 2) * bpt
        pltpu.sync_copy(shared.at[pl.ds(go * bpg + ho, bpt)], po_v)
        pltpu.sync_copy(shared.at[pl.ds((_HALF + go) * bpg + ho, bpt)], pd_v)

        def fold(i, carry):
            col = pl.multiple_of(i * _LANES, _LANES)
            po_v[pl.ds(col, _LANES)] = (
                po_v[pl.ds(col, _LANES)] + pd_v[pl.ds(col, _LANES)])
            return carry

        lax.fori_loop(0, ncho, fold, 0)
        pltpu.sync_copy(po_v, out_h.at[pl.ds(obase, bpt)])

    return run


def kernel(offense_ids, defense_ids, offense_pos, defense_pos, gamestate,
           off_embed, def_embed, bias, gs_w):
    del offense_pos, defense_pos
    B, L = offense_ids.shape
    NP = off_embed.shape[0]
    GS = gamestate.shape[1]
    tabs = jnp.concatenate(
        [off_embed.reshape(-1), def_embed.reshape(-1)]).astype(jnp.float32)
    ids = jnp.concatenate(
        [offense_ids.T.reshape(-1), defense_ids.T.reshape(-1)]
    ).astype(jnp.int32)
    g = _build_sc(B, L, NP)(tabs, ids)
    p = _build_matvec(B, GS)(
        gamestate.astype(jnp.float32),
        gs_w.astype(jnp.float32),
        bias.reshape(1, 1).astype(jnp.float32))
    return _build_combine(B)(g, p)


# speedup vs baseline: 2.5638x; 1.4660x over previous
import functools

import jax
import jax.numpy as jnp
from jax import lax
from jax.experimental import pallas as pl
from jax.experimental.pallas import tpu as pltpu
from jax.experimental.pallas import tpu_sc as plsc

_NS = 16
_LANES = 16


@functools.lru_cache(maxsize=None)
def _build_matvec(B, GS, blk=8192):
    def body(gst_ref, w_ref, b_ref, o_ref):
        o_ref[...] = jnp.sum(gst_ref[...] * w_ref[...], axis=0) + b_ref[0, 0]

    return pl.pallas_call(
        body,
        grid=(B // blk,),
        in_specs=[
            pl.BlockSpec((GS, blk), lambda i: (0, i)),
            pl.BlockSpec((GS, 1), lambda i: (0, 0)),
            pl.BlockSpec((1, 1), lambda i: (0, 0)),
        ],
        out_specs=pl.BlockSpec((blk,), lambda i: (i,)),
        out_shape=jax.ShapeDtypeStruct((B,), jnp.float32),
    )


@functools.lru_cache(maxsize=None)
def _build_combine(B):
    def body(g_ref, p_ref, o_ref):
        o_ref[...] = (g_ref[pl.ds(0, B)] + g_ref[pl.ds(B, B)]
                      + p_ref[...])

    return pl.pallas_call(
        body,
        out_shape=jax.ShapeDtypeStruct((B,), jnp.float32),
    )


@functools.lru_cache(maxsize=None)
def _build_sc(B, L, NP, nc=1):
    half = _NS // 2
    ngroups = nc * half
    bpg = B // ngroups
    nch = bpg // _LANES

    mesh = plsc.VectorSubcoreMesh(
        core_axis_name="c", subcore_axis_name="s",
        num_cores=nc, num_subcores=_NS)

    @functools.partial(
        pl.kernel,
        out_type=jax.ShapeDtypeStruct((2 * B,), jnp.float32),
        mesh=mesh,
        compiler_params=pltpu.CompilerParams(needs_layout_passes=False),
        scratch_types=[
            pltpu.VMEM((NP,), jnp.float32),
            pltpu.VMEM((L * bpg,), jnp.int32),
            pltpu.VMEM((bpg,), jnp.float32),
            pltpu.SemaphoreType.DMA,
            pltpu.SemaphoreType.DMA,
        ],
    )
    def run(otab_h, dtab_h, ids_h, out_h,
            tab_v, ids_v, psum_v, sem, sem2):
        c = lax.axis_index("c")
        s = lax.axis_index("s")
        is_off = s < half
        g = lax.rem(s, half)
        base = (c * half + g) * bpg
        obase = jnp.where(is_off, 0, B) + base

        ioffs = jnp.where(is_off, 0, L * B)

        pltpu.sync_copy(otab_h.at[pl.ds(0, 8)], psum_v.at[pl.ds(0, 8)])
        pltpu.sync_copy(dtab_h.at[pl.ds(0, 8)], psum_v.at[pl.ds(8, 8)])

        @pl.when(is_off)
        def _():
            pltpu.async_copy(otab_h, tab_v, sem)

        @pl.when(jnp.logical_not(is_off))
        def _():
            pltpu.async_copy(dtab_h, tab_v, sem)
        for l in range(L):
            pltpu.async_copy(ids_h.at[pl.ds(ioffs + l * B + base, bpg)],
                             ids_v.at[pl.ds(l * bpg, bpg)], sem2)
        for l in range(L):
            pltpu.make_async_copy(
                ids_h.at[pl.ds(ioffs + l * B + base, bpg)],
                ids_v.at[pl.ds(l * bpg, bpg)], sem2).wait()
        pltpu.make_async_copy(otab_h, tab_v, sem).wait()

        def chunk(i, carry):
            for u in range(8):
                col = pl.multiple_of(i * (8 * _LANES) + u * _LANES, _LANES)
                acc = plsc.load_gather(tab_v, [ids_v[pl.ds(col, _LANES)]])
                for l in range(1, L):
                    acc = acc + plsc.load_gather(
                        tab_v, [ids_v[pl.ds(l * bpg + col, _LANES)]])
                psum_v[pl.ds(col, _LANES)] = acc
            return carry

        lax.fori_loop(0, nch // 8, chunk, 0)

        pltpu.sync_copy(psum_v, out_h.at[pl.ds(obase, bpg)])

    return run


def kernel(offense_ids, defense_ids, offense_pos, defense_pos, gamestate,
           off_embed, def_embed, bias, gs_w):
    del offense_pos, defense_pos
    B, L = offense_ids.shape
    NP = off_embed.shape[0]
    GS = gamestate.shape[1]
    ids = jnp.concatenate(
        [offense_ids, defense_ids], axis=1).T.reshape(-1).astype(jnp.int32)
    g2 = _build_sc(B, L, NP)(
        off_embed.reshape(-1).astype(jnp.float32),
        def_embed.reshape(-1).astype(jnp.float32), ids)
    p = _build_matvec(B, GS)(
        gamestate.T.astype(jnp.float32),
        gs_w.astype(jnp.float32),
        bias.reshape(1, 1).astype(jnp.float32))
    return _build_combine(B)(g2, p)
